# baseline (device time: 172543 ns/iter reference)
import jax
import jax.numpy as jnp
from jax import lax
from jax.experimental import pallas as pl
from jax.experimental.pallas import tpu as pltpu

N_DEV = 32
M = 1024
D = 256
H = 512
N_EXP = 128
E_LOCAL = N_EXP // N_DEV
CH = M // N_DEV


def kernel(x, router_W, route_idx, expert_W, shared_W):
    def body(x_ref, router_ref, idx_ref, ew_ref, sw_ref, out_ref,
             acc_ref, rs_buf, rs_send, rs_recv, ag_send, ag_recv):
        my = lax.axis_index("i")
        left = lax.rem(my + N_DEV - 1, N_DEV)
        right = lax.rem(my + 1, N_DEV)

        xv = x_ref[:, :]
        scores = jnp.dot(xv, router_ref[:, :], preferred_element_type=jnp.float32)
        s_max = jnp.max(scores, axis=-1, keepdims=True)
        e_s = jnp.exp(scores - s_max)
        probs = e_s / jnp.sum(e_s, axis=-1, keepdims=True)
        idx = idx_ref[:, :]
        lane = lax.broadcasted_iota(jnp.int32, (M, N_EXP), 1)
        p_sel = jnp.sum(jnp.where(lane == idx, probs, 0.0), axis=-1,
                        keepdims=True)

        acc = jnp.zeros((M, H), jnp.float32)
        for j in range(E_LOCAL):
            g = my * E_LOCAL + j
            coeff = jnp.where(idx == g, p_sel, 0.0)
            y = jnp.dot(xv, ew_ref[j], preferred_element_type=jnp.float32)
            acc = acc + coeff * y
        acc_ref[:, :] = acc

        bsem = pltpu.get_barrier_semaphore()
        pl.semaphore_signal(bsem, inc=1, device_id=(left,),
                            device_id_type=pl.DeviceIdType.MESH)
        pl.semaphore_signal(bsem, inc=1, device_id=(right,),
                            device_id_type=pl.DeviceIdType.MESH)
        pl.semaphore_wait(bsem, 2)

        for s in range(N_DEV - 1):
            c_send = lax.rem(my - s + N_DEV, N_DEV)
            c_recv = lax.rem(my - s - 1 + N_DEV, N_DEV)
            rdma = pltpu.make_async_remote_copy(
                src_ref=acc_ref.at[pl.ds(c_send * CH, CH), :],
                dst_ref=rs_buf.at[s],
                send_sem=rs_send.at[s],
                recv_sem=rs_recv.at[s],
                device_id=(right,),
                device_id_type=pl.DeviceIdType.MESH,
            )
            rdma.start()
            rdma.wait()
            rows = pl.ds(c_recv * CH, CH)
            acc_ref[rows, :] = acc_ref[rows, :] + rs_buf[s]

        c_mine = lax.rem(my + 1, N_DEV)
        rows = pl.ds(c_mine * CH, CH)
        shared = jnp.dot(x_ref[rows, :], sw_ref[:, :],
                         preferred_element_type=jnp.float32)
        out_ref[rows, :] = acc_ref[rows, :] + shared

        for t in range(N_DEV - 1):
            c_send = lax.rem(my + 1 - t + N_DEV, N_DEV)
            sl = pl.ds(c_send * CH, CH)
            rdma = pltpu.make_async_remote_copy(
                src_ref=out_ref.at[sl, :],
                dst_ref=out_ref.at[sl, :],
                send_sem=ag_send.at[t],
                recv_sem=ag_recv.at[t],
                device_id=(right,),
                device_id_type=pl.DeviceIdType.MESH,
            )
            rdma.start()
            rdma.wait()

    return pl.pallas_call(
        body,
        out_shape=jax.ShapeDtypeStruct((M, H), jnp.float32),
        in_specs=[pl.BlockSpec(memory_space=pltpu.VMEM)] * 5,
        out_specs=pl.BlockSpec(memory_space=pltpu.VMEM),
        scratch_shapes=[
            pltpu.VMEM((M, H), jnp.float32),
            pltpu.VMEM((N_DEV - 1, CH, H), jnp.float32),
            pltpu.SemaphoreType.DMA((N_DEV - 1,)),
            pltpu.SemaphoreType.DMA((N_DEV - 1,)),
            pltpu.SemaphoreType.DMA((N_DEV - 1,)),
            pltpu.SemaphoreType.DMA((N_DEV - 1,)),
        ],
        compiler_params=pltpu.CompilerParams(collective_id=0),
    )(x, router_W, route_idx, expert_W, shared_W)


# device time: 70099 ns/iter; 2.4614x vs baseline; 2.4614x over previous
import jax
import jax.numpy as jnp
from jax import lax
from jax.experimental import pallas as pl
from jax.experimental.pallas import tpu as pltpu

N_DEV = 32
M = 1024
D = 256
H = 512
N_EXP = 128
E_LOCAL = N_EXP // N_DEV
CH = M // N_DEV

_MESH = pl.DeviceIdType.MESH


def kernel(x, router_W, route_idx, expert_W, shared_W):
    def body(x_ref, router_ref, idx_ref, ew_ref, sw_ref, out_ref,
             acc_ref, rs_buf, r_send, r_recv, b_send, b_recv):
        my = lax.axis_index("i")

        xv = x_ref[:, :]
        scores = jnp.dot(xv, router_ref[:, :], preferred_element_type=jnp.float32)
        s_max = jnp.max(scores, axis=-1, keepdims=True)
        e_s = jnp.exp(scores - s_max)
        probs = e_s / jnp.sum(e_s, axis=-1, keepdims=True)
        idx = idx_ref[:, :]
        lane = lax.broadcasted_iota(jnp.int32, (M, N_EXP), 1)
        p_sel = jnp.sum(jnp.where(lane == idx, probs, 0.0), axis=-1,
                        keepdims=True)

        acc = jnp.zeros((M, H), jnp.float32)
        for j in range(E_LOCAL):
            g = my * E_LOCAL + j
            coeff = jnp.where(idx == g, p_sel, 0.0)
            y = jnp.dot(xv, ew_ref[j], preferred_element_type=jnp.float32)
            acc = acc + coeff * y
        acc_ref[:, :] = acc

        bsem = pltpu.get_barrier_semaphore()
        for k in range(1, N_DEV):
            peer = lax.rem(my + k, N_DEV)
            pl.semaphore_signal(bsem, inc=1, device_id=(peer,),
                                device_id_type=_MESH)
        pl.semaphore_wait(bsem, N_DEV - 1)

        for k in range(1, N_DEV):
            t = lax.rem(my + k, N_DEV)
            pltpu.make_async_remote_copy(
                src_ref=acc_ref.at[pl.ds(t * CH, CH), :],
                dst_ref=rs_buf.at[k - 1],
                send_sem=r_send.at[k - 1],
                recv_sem=r_recv.at[k - 1],
                device_id=(t,),
                device_id_type=_MESH,
            ).start()
        for k in range(1, N_DEV):
            pltpu.make_async_remote_copy(
                src_ref=acc_ref.at[pl.ds(0, CH), :],
                dst_ref=rs_buf.at[k - 1],
                send_sem=r_send.at[k - 1],
                recv_sem=r_recv.at[k - 1],
                device_id=(my,),
                device_id_type=_MESH,
            ).wait_recv()

        mine = pl.ds(my * CH, CH)
        red = acc_ref[mine, :] + jnp.sum(rs_buf[:, :, :], axis=0)
        shared = jnp.dot(x_ref[mine, :], sw_ref[:, :],
                         preferred_element_type=jnp.float32)
        out_ref[mine, :] = red + shared

        for k in range(1, N_DEV):
            t = lax.rem(my + k, N_DEV)
            pltpu.make_async_remote_copy(
                src_ref=out_ref.at[mine, :],
                dst_ref=out_ref.at[mine, :],
                send_sem=b_send.at[k - 1],
                recv_sem=b_recv.at[k - 1],
                device_id=(t,),
                device_id_type=_MESH,
            ).start()
        for k in range(1, N_DEV):
            src_d = lax.rem(my - k + N_DEV, N_DEV)
            pltpu.make_async_remote_copy(
                src_ref=out_ref.at[mine, :],
                dst_ref=out_ref.at[pl.ds(src_d * CH, CH), :],
                send_sem=b_send.at[k - 1],
                recv_sem=b_recv.at[k - 1],
                device_id=(my,),
                device_id_type=_MESH,
            ).wait_recv()

        for k in range(1, N_DEV):
            pltpu.make_async_remote_copy(
                src_ref=acc_ref.at[pl.ds(0, CH), :],
                dst_ref=rs_buf.at[k - 1],
                send_sem=r_send.at[k - 1],
                recv_sem=r_recv.at[k - 1],
                device_id=(my,),
                device_id_type=_MESH,
            ).wait_send()
            pltpu.make_async_remote_copy(
                src_ref=out_ref.at[mine, :],
                dst_ref=rs_buf.at[k - 1],
                send_sem=b_send.at[k - 1],
                recv_sem=b_recv.at[k - 1],
                device_id=(my,),
                device_id_type=_MESH,
            ).wait_send()

    return pl.pallas_call(
        body,
        out_shape=jax.ShapeDtypeStruct((M, H), jnp.float32),
        in_specs=[pl.BlockSpec(memory_space=pltpu.VMEM)] * 5,
        out_specs=pl.BlockSpec(memory_space=pltpu.VMEM),
        scratch_shapes=[
            pltpu.VMEM((M, H), jnp.float32),
            pltpu.VMEM((N_DEV - 1, CH, H), jnp.float32),
            pltpu.SemaphoreType.DMA((N_DEV - 1,)),
            pltpu.SemaphoreType.DMA((N_DEV - 1,)),
            pltpu.SemaphoreType.DMA((N_DEV - 1,)),
            pltpu.SemaphoreType.DMA((N_DEV - 1,)),
        ],
        compiler_params=pltpu.CompilerParams(collective_id=0),
    )(x, router_W, route_idx, expert_W, shared_W)
